# Initial kernel scaffold; baseline (speedup 1.0000x reference)
import jax
import jax.numpy as jnp
from jax.experimental import pallas as pl


def kernel(features, edge_index, perm, labels, W1, b1, W2, b2, Wm, bm):
    raise NotImplementedError("write your pallas kernel here")


# speedup vs baseline: 1.9400x; 1.9400x over previous
import functools

import jax
import jax.numpy as jnp
from jax import lax
from jax.experimental import pallas as pl
from jax.experimental.pallas import tpu as pltpu
from jax.experimental.pallas import tpu_sc as plsc

_N = 10000
_E = 320000
_D = 128
_HD = _D // 2
_NC = 2
_NS = 16
_NW = _NC * _NS
_CB = 128
_CH = 80
_EP = _NW * _CH * _CB
_NPAD = 10112
_RPS = _NPAD // _NS
_PPW = 320
_PN = _NW * _PPW
_BLK = 1000
_GRID = _N // _BLK


def _deg_body(src_hbm, dst_hbm, perm_hbm, feat_hbm, ones_hbm, zeros16_hbm,
              dout_hbm, din_hbm, fperm_hbm,
              sidx, didx, ones_v, pidx, frows, acco, acci, sem):
    cid = lax.axis_index("c")
    sid = lax.axis_index("s")
    wid = sid * _NC + cid
    pltpu.sync_copy(src_hbm.at[wid], sidx)
    pltpu.sync_copy(dst_hbm.at[wid], didx)
    pltpu.sync_copy(ones_hbm, ones_v)
    pltpu.sync_copy(zeros16_hbm, acco.at[pl.ds(sid * _RPS, _RPS)])
    pltpu.sync_copy(zeros16_hbm, acci.at[pl.ds(sid * _RPS, _RPS)])
    pltpu.sync_copy(perm_hbm.at[wid], pidx)
    g0 = pltpu.async_copy(feat_hbm.at[pidx.at[pl.ds(0, 128)]],
                          frows.at[pl.ds(0, 128)], sem)
    g1 = pltpu.async_copy(feat_hbm.at[pidx.at[pl.ds(128, 128)]],
                          frows.at[pl.ds(128, 128)], sem)
    g2 = pltpu.async_copy(feat_hbm.at[pidx.at[pl.ds(256, 64)]],
                          frows.at[pl.ds(256, 64)], sem)
    plsc.subcore_barrier()
    for c in range(_CH):
        pltpu.sync_copy(ones_v, acco.at[sidx.at[c]], add=True)
        pltpu.sync_copy(ones_v, acci.at[didx.at[c]], add=True)
    plsc.subcore_barrier()
    pltpu.sync_copy(acco.at[pl.ds(sid * _RPS, _RPS)],
                    dout_hbm.at[cid, pl.ds(sid * _RPS, _RPS)])
    pltpu.sync_copy(acci.at[pl.ds(sid * _RPS, _RPS)],
                    din_hbm.at[cid, pl.ds(sid * _RPS, _RPS)])
    g0.wait()
    g1.wait()
    g2.wait()
    pltpu.sync_copy(frows, fperm_hbm.at[pl.ds(wid * _PPW, _PPW)])


@functools.cache
def _deg():
    return pl.kernel(
        _deg_body,
        out_type=(
            jax.ShapeDtypeStruct((_NC, _NPAD, 16), jnp.float32),
            jax.ShapeDtypeStruct((_NC, _NPAD, 16), jnp.float32),
            jax.ShapeDtypeStruct((_PN, _D), jnp.float32),
        ),
        mesh=plsc.VectorSubcoreMesh(core_axis_name="c", subcore_axis_name="s"),
        compiler_params=pltpu.CompilerParams(use_tc_tiling_on_sc=False),
        scratch_types=[
            pltpu.VMEM((_CH, _CB), jnp.int32),
            pltpu.VMEM((_CH, _CB), jnp.int32),
            pltpu.VMEM((_CB, 16), jnp.float32),
            pltpu.VMEM((_PPW,), jnp.int32),
            pltpu.VMEM((_PPW, _D), jnp.float32),
            pltpu.VMEM_SHARED((_NPAD, 16), jnp.float32),
            pltpu.VMEM_SHARED((_NPAD, 16), jnp.float32),
            pltpu.SemaphoreType.DMA,
        ],
    )


def _spmm_body(xlo_hbm, xhi_hbm, src_hbm, dst_hbm, zeros_hbm,
               outlo_hbm, outhi_hbm,
               sidx, didx, rows, acc, sem0, sem1):
    cid = lax.axis_index("c")
    sid = lax.axis_index("s")
    wid = sid * _NC + cid
    pltpu.sync_copy(src_hbm.at[wid], sidx)
    pltpu.sync_copy(dst_hbm.at[wid], didx)
    sems = (sem0, sem1)
    for x_hbm, out_hbm in ((xlo_hbm, outlo_hbm), (xhi_hbm, outhi_hbm)):
        pltpu.sync_copy(zeros_hbm, acc.at[pl.ds(sid * _RPS, _RPS)])
        plsc.subcore_barrier()
        handles = [None, None]
        handles[0] = pltpu.async_copy(x_hbm.at[sidx.at[0]], rows.at[0], sem0)
        for c in range(_CH):
            b = c & 1
            if c + 1 < _CH:
                nb = b ^ 1
                handles[nb] = pltpu.async_copy(x_hbm.at[sidx.at[c + 1]],
                                               rows.at[nb], sems[nb])
            handles[b].wait()
            pltpu.sync_copy(rows.at[b], acc.at[didx.at[c]], add=True)
        plsc.subcore_barrier()
        pltpu.sync_copy(acc.at[pl.ds(sid * _RPS, _RPS)],
                        out_hbm.at[cid, pl.ds(sid * _RPS, _RPS)])


@functools.cache
def _spmm():
    return pl.kernel(
        _spmm_body,
        out_type=(
            jax.ShapeDtypeStruct((_NC, _NPAD, _HD), jnp.float32),
            jax.ShapeDtypeStruct((_NC, _NPAD, _HD), jnp.float32),
        ),
        mesh=plsc.VectorSubcoreMesh(core_axis_name="c", subcore_axis_name="s"),
        compiler_params=pltpu.CompilerParams(use_tc_tiling_on_sc=False),
        scratch_types=[
            pltpu.VMEM((_CH, _CB), jnp.int32),
            pltpu.VMEM((_CH, _CB), jnp.int32),
            pltpu.VMEM((2, _CB, _HD), jnp.float32),
            pltpu.VMEM_SHARED((_NPAD, _HD), jnp.float32),
            pltpu.SemaphoreType.DMA,
            pltpu.SemaphoreType.DMA,
        ],
    )


def _norm(p_ref):
    cnt = p_ref[0, :, 0:1] + p_ref[1, :, 0:1]
    return lax.rsqrt(jnp.maximum(cnt, 1.0))


def _prep_body(feat_ref, fperm_ref, dop_ref,
               x1lo_ref, x1hi_ref, x2lo_ref, x2hi_ref):
    no = _norm(dop_ref)
    x1 = feat_ref[...] * no
    x2 = fperm_ref[...] * no
    x1lo_ref[...] = x1[:, :_HD]
    x1hi_ref[...] = x1[:, _HD:]
    x2lo_ref[...] = x2[:, :_HD]
    x2hi_ref[...] = x2[:, _HD:]


_prep = pl.pallas_call(
    _prep_body,
    grid=(_GRID,),
    in_specs=[
        pl.BlockSpec((_BLK, _D), lambda i: (i, 0)),
        pl.BlockSpec((_BLK, _D), lambda i: (i, 0)),
        pl.BlockSpec((_NC, _BLK, 16), lambda i: (0, i, 0)),
    ],
    out_specs=[pl.BlockSpec((_BLK, _HD), lambda i: (i, 0))] * 4,
    out_shape=[jax.ShapeDtypeStruct((_NPAD, _HD), jnp.float32)] * 4,
)


def _agg_cat(alo_ref, ahi_ref):
    return jnp.concatenate(
        [alo_ref[0, :, :] + alo_ref[1, :, :],
         ahi_ref[0, :, :] + ahi_ref[1, :, :]], axis=1)


def _layer_body(alo_ref, ahi_ref, dop_ref, dip_ref, w_ref, b_ref,
                ylo_ref, yhi_ref):
    agg = _agg_cat(alo_ref, ahi_ref)
    ni = _norm(dip_ref)
    no = _norm(dop_ref)
    z = jnp.dot(agg, w_ref[...], preferred_element_type=jnp.float32)
    h = jnp.maximum(z * ni + b_ref[...], 0.0)
    y = h * no
    ylo_ref[...] = y[:, :_HD]
    yhi_ref[...] = y[:, _HD:]


_layer = pl.pallas_call(
    _layer_body,
    grid=(_GRID,),
    in_specs=[
        pl.BlockSpec((_NC, _BLK, _HD), lambda i: (0, i, 0)),
        pl.BlockSpec((_NC, _BLK, _HD), lambda i: (0, i, 0)),
        pl.BlockSpec((_NC, _BLK, 16), lambda i: (0, i, 0)),
        pl.BlockSpec((_NC, _BLK, 16), lambda i: (0, i, 0)),
        pl.BlockSpec((_D, _D), lambda i: (0, 0)),
        pl.BlockSpec((1, _D), lambda i: (0, 0)),
    ],
    out_specs=[pl.BlockSpec((_BLK, _HD), lambda i: (i, 0))] * 2,
    out_shape=[jax.ShapeDtypeStruct((_NPAD, _HD), jnp.float32)] * 2,
)


def _final_body(alo1_ref, ahi1_ref, alo2_ref, ahi2_ref, dip_ref,
                w2_ref, b2_ref, wm_ref, bm_ref, lab_ref, out_ref):
    i = pl.program_id(0)

    @pl.when(i == 0)
    def _():
        out_ref[0, 0] = 0.0

    ni = _norm(dip_ref)
    wmv = jnp.sum(wm_ref[...], axis=1, keepdims=True)
    bms = jnp.sum(bm_ref[...])
    total = 0.0
    for e, (alo, ahi) in ((0, (alo1_ref, ahi1_ref)),
                          (1, (alo2_ref, ahi2_ref))):
        agg = _agg_cat(alo, ahi)
        z = jnp.dot(agg, w2_ref[...], preferred_element_type=jnp.float32)
        h = jnp.maximum(z * ni + b2_ref[...], 0.0)
        s = jnp.dot(h, wmv, preferred_element_type=jnp.float32) + bms
        l = lab_ref[e, :, :]
        bce = jnp.maximum(s, 0.0) - s * l + jnp.log(1.0 + jnp.exp(-jnp.abs(s)))
        total = total + jnp.sum(bce)
    out_ref[0, 0] = out_ref[0, 0] + total / (2.0 * _N)


_final = pl.pallas_call(
    _final_body,
    grid=(_GRID,),
    in_specs=[
        pl.BlockSpec((_NC, _BLK, _HD), lambda i: (0, i, 0)),
        pl.BlockSpec((_NC, _BLK, _HD), lambda i: (0, i, 0)),
        pl.BlockSpec((_NC, _BLK, _HD), lambda i: (0, i, 0)),
        pl.BlockSpec((_NC, _BLK, _HD), lambda i: (0, i, 0)),
        pl.BlockSpec((_NC, _BLK, 16), lambda i: (0, i, 0)),
        pl.BlockSpec((_D, _D), lambda i: (0, 0)),
        pl.BlockSpec((1, _D), lambda i: (0, 0)),
        pl.BlockSpec((_D, _D), lambda i: (0, 0)),
        pl.BlockSpec((1, _D), lambda i: (0, 0)),
        pl.BlockSpec((2, _BLK, 1), lambda i: (0, i, 0)),
    ],
    out_specs=pl.BlockSpec(memory_space=pltpu.SMEM),
    out_shape=jax.ShapeDtypeStruct((1, 1), jnp.float32),
)


def kernel(features, edge_index, perm, labels, W1, b1, W2, b2, Wm, bm):
    src = edge_index[0]
    dst = edge_index[1]
    padv = jnp.full((_EP - _E,), _N, jnp.int32)
    src3 = jnp.concatenate([src, padv]).reshape(_NW, _CH, _CB)
    dst3 = jnp.concatenate([dst, padv]).reshape(_NW, _CH, _CB)
    perm2 = jnp.concatenate(
        [perm, jnp.zeros((_PN - _N,), jnp.int32)]).reshape(_NW, _PPW)
    zeros64 = jnp.zeros((_RPS, _HD), jnp.float32)
    zeros16 = jnp.zeros((_RPS, 16), jnp.float32)
    ones16 = jnp.ones((_CB, 16), jnp.float32)
    b1r = b1.reshape(1, _D)
    b2r = b2.reshape(1, _D)
    bmr = bm.reshape(1, _D)
    lab2 = labels.reshape(2, _N, 1)

    deg_k = _deg()
    spmm_k = _spmm()
    dout_p, din_p, fperm = deg_k(src3, dst3, perm2, features, ones16, zeros16)
    x1lo, x1hi, x2lo, x2hi = _prep(features, fperm[:_N], dout_p)
    a1lo, a1hi = spmm_k(x1lo, x1hi, src3, dst3, zeros64)
    a2lo, a2hi = spmm_k(x2lo, x2hi, src3, dst3, zeros64)
    y1lo, y1hi = _layer(a1lo, a1hi, dout_p, din_p, W1, b1r)
    y2lo, y2hi = _layer(a2lo, a2hi, dout_p, din_p, W1, b1r)
    g1lo, g1hi = spmm_k(y1lo, y1hi, src3, dst3, zeros64)
    g2lo, g2hi = spmm_k(y2lo, y2hi, src3, dst3, zeros64)
    out = _final(g1lo, g1hi, g2lo, g2hi, din_p, W2, b2r, Wm, bmr, lab2)
    return out[0, 0]


# speedup vs baseline: 5.2377x; 2.6999x over previous
import functools

import jax
import jax.numpy as jnp
from jax import lax
from jax.experimental import pallas as pl
from jax.experimental.pallas import tpu as pltpu
from jax.experimental.pallas import tpu_sc as plsc

_N = 10000
_E = 320000
_D = 128
_HD = _D // 2
_NC = 2
_NS = 16
_NW = _NC * _NS
_CB = 128
_CH = 80
_EP = _NW * _CH * _CB
_NPAD = 10112
_RPS = _NPAD // _NS
_PPW = 320
_PN = _NW * _PPW
_BLK = 1000
_GRID = _N // _BLK


def _deg_body(src_hbm, dst_hbm, perm_hbm, feat_hbm, ones_hbm, zeros16_hbm,
              dout_hbm, din_hbm, fperm_hbm,
              sidx, didx, ones_v, pidx, frows, acco, acci, sem):
    cid = lax.axis_index("c")
    sid = lax.axis_index("s")
    wid = sid * _NC + cid
    pltpu.sync_copy(src_hbm.at[wid], sidx)
    pltpu.sync_copy(dst_hbm.at[wid], didx)
    pltpu.sync_copy(ones_hbm, ones_v)
    pltpu.sync_copy(zeros16_hbm, acco.at[pl.ds(sid * _RPS, _RPS)])
    pltpu.sync_copy(zeros16_hbm, acci.at[pl.ds(sid * _RPS, _RPS)])
    pltpu.sync_copy(perm_hbm.at[wid], pidx)
    g0 = pltpu.async_copy(feat_hbm.at[pidx.at[pl.ds(0, 128)]],
                          frows.at[pl.ds(0, 128)], sem)
    g1 = pltpu.async_copy(feat_hbm.at[pidx.at[pl.ds(128, 128)]],
                          frows.at[pl.ds(128, 128)], sem)
    g2 = pltpu.async_copy(feat_hbm.at[pidx.at[pl.ds(256, 64)]],
                          frows.at[pl.ds(256, 64)], sem)
    plsc.subcore_barrier()
    for c in range(_CH):
        pltpu.sync_copy(ones_v, acco.at[sidx.at[c]], add=True)
        pltpu.sync_copy(ones_v, acci.at[didx.at[c]], add=True)
    plsc.subcore_barrier()
    pltpu.sync_copy(acco.at[pl.ds(sid * _RPS, _RPS)],
                    dout_hbm.at[cid, pl.ds(sid * _RPS, _RPS)])
    pltpu.sync_copy(acci.at[pl.ds(sid * _RPS, _RPS)],
                    din_hbm.at[cid, pl.ds(sid * _RPS, _RPS)])
    g0.wait()
    g1.wait()
    g2.wait()
    pltpu.sync_copy(frows, fperm_hbm.at[pl.ds(wid * _PPW, _PPW)])


@functools.cache
def _deg():
    return pl.kernel(
        _deg_body,
        out_type=(
            jax.ShapeDtypeStruct((_NC, _NPAD, 16), jnp.float32),
            jax.ShapeDtypeStruct((_NC, _NPAD, 16), jnp.float32),
            jax.ShapeDtypeStruct((_PN, _D), jnp.float32),
        ),
        mesh=plsc.VectorSubcoreMesh(core_axis_name="c", subcore_axis_name="s"),
        compiler_params=pltpu.CompilerParams(use_tc_tiling_on_sc=False),
        scratch_types=[
            pltpu.VMEM((_CH, _CB), jnp.int32),
            pltpu.VMEM((_CH, _CB), jnp.int32),
            pltpu.VMEM((_CB, 16), jnp.float32),
            pltpu.VMEM((_PPW,), jnp.int32),
            pltpu.VMEM((_PPW, _D), jnp.float32),
            pltpu.VMEM_SHARED((_NPAD, 16), jnp.float32),
            pltpu.VMEM_SHARED((_NPAD, 16), jnp.float32),
            pltpu.SemaphoreType.DMA,
        ],
    )


def _spmm_body(xlo_hbm, xhi_hbm, src_hbm, dst_hbm, zeros_hbm,
               outlo_hbm, outhi_hbm,
               sidx, didx, rows, acc, sem0, sem1):
    cid = lax.axis_index("c")
    sid = lax.axis_index("s")
    wid = sid * _NC + cid
    pltpu.sync_copy(src_hbm.at[wid], sidx)
    pltpu.sync_copy(dst_hbm.at[wid], didx)
    sems = (sem0, sem1)
    for x_hbm, out_hbm in ((xlo_hbm, outlo_hbm), (xhi_hbm, outhi_hbm)):
        pltpu.sync_copy(zeros_hbm, acc.at[pl.ds(sid * _RPS, _RPS)])
        plsc.subcore_barrier()
        handles = [None, None]
        handles[0] = pltpu.async_copy(x_hbm.at[sidx.at[0]], rows.at[0], sem0)
        for c in range(_CH):
            b = c & 1
            if c + 1 < _CH:
                nb = b ^ 1
                handles[nb] = pltpu.async_copy(x_hbm.at[sidx.at[c + 1]],
                                               rows.at[nb], sems[nb])
            handles[b].wait()
            pltpu.sync_copy(rows.at[b], acc.at[didx.at[c]], add=True)
        plsc.subcore_barrier()
        pltpu.sync_copy(acc.at[pl.ds(sid * _RPS, _RPS)],
                        out_hbm.at[cid, pl.ds(sid * _RPS, _RPS)])


@functools.cache
def _spmm():
    return pl.kernel(
        _spmm_body,
        out_type=(
            jax.ShapeDtypeStruct((_NC, _NPAD, _HD), jnp.float32),
            jax.ShapeDtypeStruct((_NC, _NPAD, _HD), jnp.float32),
        ),
        mesh=plsc.VectorSubcoreMesh(core_axis_name="c", subcore_axis_name="s"),
        compiler_params=pltpu.CompilerParams(use_tc_tiling_on_sc=False),
        scratch_types=[
            pltpu.VMEM((_CH, _CB), jnp.int32),
            pltpu.VMEM((_CH, _CB), jnp.int32),
            pltpu.VMEM((2, _CB, _HD), jnp.float32),
            pltpu.VMEM_SHARED((_NPAD, _HD), jnp.float32),
            pltpu.SemaphoreType.DMA,
            pltpu.SemaphoreType.DMA,
        ],
    )


def _norm(p_ref):
    cnt = p_ref[0, :, 0:1] + p_ref[1, :, 0:1]
    return lax.rsqrt(jnp.maximum(cnt, 1.0))


def _prep_body(feat_ref, fperm_ref, dop_ref,
               x1lo_ref, x1hi_ref, x2lo_ref, x2hi_ref):
    no = _norm(dop_ref)
    x1 = feat_ref[...] * no
    x2 = fperm_ref[...] * no
    x1lo_ref[...] = x1[:, :_HD]
    x1hi_ref[...] = x1[:, _HD:]
    x2lo_ref[...] = x2[:, :_HD]
    x2hi_ref[...] = x2[:, _HD:]


_prep = pl.pallas_call(
    _prep_body,
    grid=(_GRID,),
    in_specs=[
        pl.BlockSpec((_BLK, _D), lambda i: (i, 0)),
        pl.BlockSpec((_BLK, _D), lambda i: (i, 0)),
        pl.BlockSpec((_NC, _BLK, 16), lambda i: (0, i, 0)),
    ],
    out_specs=[pl.BlockSpec((_BLK, _HD), lambda i: (i, 0))] * 4,
    out_shape=[jax.ShapeDtypeStruct((_NPAD, _HD), jnp.float32)] * 4,
)


def _agg_cat(alo_ref, ahi_ref):
    return jnp.concatenate(
        [alo_ref[0, :, :] + alo_ref[1, :, :],
         ahi_ref[0, :, :] + ahi_ref[1, :, :]], axis=1)


def _layer_body(alo_ref, ahi_ref, dop_ref, dip_ref, w_ref, b_ref,
                ylo_ref, yhi_ref):
    agg = _agg_cat(alo_ref, ahi_ref)
    ni = _norm(dip_ref)
    no = _norm(dop_ref)
    z = jnp.dot(agg, w_ref[...], preferred_element_type=jnp.float32)
    h = jnp.maximum(z * ni + b_ref[...], 0.0)
    y = h * no
    ylo_ref[...] = y[:, :_HD]
    yhi_ref[...] = y[:, _HD:]


_layer = pl.pallas_call(
    _layer_body,
    grid=(_GRID,),
    in_specs=[
        pl.BlockSpec((_NC, _BLK, _HD), lambda i: (0, i, 0)),
        pl.BlockSpec((_NC, _BLK, _HD), lambda i: (0, i, 0)),
        pl.BlockSpec((_NC, _BLK, 16), lambda i: (0, i, 0)),
        pl.BlockSpec((_NC, _BLK, 16), lambda i: (0, i, 0)),
        pl.BlockSpec((_D, _D), lambda i: (0, 0)),
        pl.BlockSpec((1, _D), lambda i: (0, 0)),
    ],
    out_specs=[pl.BlockSpec((_BLK, _HD), lambda i: (i, 0))] * 2,
    out_shape=[jax.ShapeDtypeStruct((_NPAD, _HD), jnp.float32)] * 2,
)


def _final_body(alo1_ref, ahi1_ref, alo2_ref, ahi2_ref, dip_ref,
                w2_ref, b2_ref, wm_ref, bm_ref, lab_ref, out_ref):
    i = pl.program_id(0)

    @pl.when(i == 0)
    def _():
        out_ref[0, 0] = 0.0

    ni = _norm(dip_ref)
    wmv = jnp.sum(wm_ref[...], axis=1, keepdims=True)
    bms = jnp.sum(bm_ref[...])
    total = 0.0
    for e, (alo, ahi) in ((0, (alo1_ref, ahi1_ref)),
                          (1, (alo2_ref, ahi2_ref))):
        agg = _agg_cat(alo, ahi)
        z = jnp.dot(agg, w2_ref[...], preferred_element_type=jnp.float32)
        h = jnp.maximum(z * ni + b2_ref[...], 0.0)
        s = jnp.dot(h, wmv, preferred_element_type=jnp.float32) + bms
        l = lab_ref[e, :, :]
        bce = jnp.maximum(s, 0.0) - s * l + jnp.log(1.0 + jnp.exp(-jnp.abs(s)))
        total = total + jnp.sum(bce)
    out_ref[0, 0] = out_ref[0, 0] + total / (2.0 * _N)


_final = pl.pallas_call(
    _final_body,
    grid=(_GRID,),
    in_specs=[
        pl.BlockSpec((_NC, _BLK, _HD), lambda i: (0, i, 0)),
        pl.BlockSpec((_NC, _BLK, _HD), lambda i: (0, i, 0)),
        pl.BlockSpec((_NC, _BLK, _HD), lambda i: (0, i, 0)),
        pl.BlockSpec((_NC, _BLK, _HD), lambda i: (0, i, 0)),
        pl.BlockSpec((_NC, _BLK, 16), lambda i: (0, i, 0)),
        pl.BlockSpec((_D, _D), lambda i: (0, 0)),
        pl.BlockSpec((1, _D), lambda i: (0, 0)),
        pl.BlockSpec((_D, _D), lambda i: (0, 0)),
        pl.BlockSpec((1, _D), lambda i: (0, 0)),
        pl.BlockSpec((2, _BLK, 1), lambda i: (0, i, 0)),
    ],
    out_specs=pl.BlockSpec(memory_space=pltpu.SMEM),
    out_shape=jax.ShapeDtypeStruct((1, 1), jnp.float32),
)


def kernel(features, edge_index, perm, labels, W1, b1, W2, b2, Wm, bm):
    src = edge_index[0]
    dst = edge_index[1]
    ppw = (_EP - _E) // _NW
    padv = _N + (jnp.arange(_NW * ppw, dtype=jnp.int32) % (_NPAD - _N))
    padv = padv.reshape(_NW, ppw)
    src3 = jnp.concatenate(
        [src.reshape(_NW, _E // _NW), padv], axis=1).reshape(_NW, _CH, _CB)
    dst3 = jnp.concatenate(
        [dst.reshape(_NW, _E // _NW), padv], axis=1).reshape(_NW, _CH, _CB)
    perm2 = jnp.concatenate(
        [perm, jnp.zeros((_PN - _N,), jnp.int32)]).reshape(_NW, _PPW)
    zeros64 = jnp.zeros((_RPS, _HD), jnp.float32)
    zeros16 = jnp.zeros((_RPS, 16), jnp.float32)
    ones16 = jnp.ones((_CB, 16), jnp.float32)
    b1r = b1.reshape(1, _D)
    b2r = b2.reshape(1, _D)
    bmr = bm.reshape(1, _D)
    lab2 = labels.reshape(2, _N, 1)

    deg_k = _deg()
    spmm_k = _spmm()
    dout_p, din_p, fperm = deg_k(src3, dst3, perm2, features, ones16, zeros16)
    x1lo, x1hi, x2lo, x2hi = _prep(features, fperm[:_N], dout_p)
    a1lo, a1hi = spmm_k(x1lo, x1hi, src3, dst3, zeros64)
    a2lo, a2hi = spmm_k(x2lo, x2hi, src3, dst3, zeros64)
    y1lo, y1hi = _layer(a1lo, a1hi, dout_p, din_p, W1, b1r)
    y2lo, y2hi = _layer(a2lo, a2hi, dout_p, din_p, W1, b1r)
    g1lo, g1hi = spmm_k(y1lo, y1hi, src3, dst3, zeros64)
    g2lo, g2hi = spmm_k(y2lo, y2hi, src3, dst3, zeros64)
    out = _final(g1lo, g1hi, g2lo, g2hi, din_p, W2, b2r, Wm, bmr, lab2)
    return out[0, 0]


# speedup vs baseline: 6.2813x; 1.1993x over previous
import functools

import jax
import jax.numpy as jnp
from jax import lax
from jax.experimental import pallas as pl
from jax.experimental.pallas import tpu as pltpu
from jax.experimental.pallas import tpu_sc as plsc

_N = 10000
_E = 320000
_D = 128
_NC = 2
_NS = 16
_NW = _NC * _NS
_CB = 125
_CH = 80
_CH2 = 160
_RPS = _N // _NS
_PPW = 320
_PN = _NW * _PPW
_NB = 4
_HD = _D // 2
_BLK = 1000
_GRID = _N // _BLK

_SC_PARAMS = pltpu.CompilerParams(use_tc_tiling_on_sc=False,
                                  internal_scratch_in_bytes=0)


def _deg_body(src_hbm, dst_hbm, perm_hbm, feat_hbm, ones_hbm, zeros16_hbm,
              dout_hbm, din_hbm, fperm_hbm,
              sidx, didx, ones_v, pidx, frows, acco, acci, sem):
    cid = lax.axis_index("c")
    sid = lax.axis_index("s")
    wid = sid * _NC + cid
    pltpu.sync_copy(src_hbm.at[wid], sidx)
    pltpu.sync_copy(dst_hbm.at[wid], didx)
    pltpu.sync_copy(ones_hbm, ones_v)
    pltpu.sync_copy(zeros16_hbm, acco.at[pl.ds(sid * _RPS, _RPS)])
    pltpu.sync_copy(zeros16_hbm, acci.at[pl.ds(sid * _RPS, _RPS)])
    pltpu.sync_copy(perm_hbm.at[wid], pidx)
    g0 = pltpu.async_copy(feat_hbm.at[pidx.at[pl.ds(0, 128)]],
                          frows.at[pl.ds(0, 128)], sem)
    g1 = pltpu.async_copy(feat_hbm.at[pidx.at[pl.ds(128, 128)]],
                          frows.at[pl.ds(128, 128)], sem)
    g2 = pltpu.async_copy(feat_hbm.at[pidx.at[pl.ds(256, 64)]],
                          frows.at[pl.ds(256, 64)], sem)
    plsc.subcore_barrier()
    for c in range(_CH):
        pltpu.sync_copy(ones_v, acco.at[sidx.at[c]], add=True)
        pltpu.sync_copy(ones_v, acci.at[didx.at[c]], add=True)
    plsc.subcore_barrier()
    pltpu.sync_copy(acco.at[pl.ds(sid * _RPS, _RPS)],
                    dout_hbm.at[cid, pl.ds(sid * _RPS, _RPS)])
    pltpu.sync_copy(acci.at[pl.ds(sid * _RPS, _RPS)],
                    din_hbm.at[cid, pl.ds(sid * _RPS, _RPS)])
    g0.wait()
    g1.wait()
    g2.wait()
    pltpu.sync_copy(frows, fperm_hbm.at[pl.ds(wid * _PPW, _PPW)])


@functools.cache
def _deg():
    return pl.kernel(
        _deg_body,
        out_type=(
            jax.ShapeDtypeStruct((_NC, _N, 16), jnp.float32),
            jax.ShapeDtypeStruct((_NC, _N, 16), jnp.float32),
            jax.ShapeDtypeStruct((_PN, _D), jnp.float32),
        ),
        mesh=plsc.VectorSubcoreMesh(core_axis_name="c", subcore_axis_name="s"),
        compiler_params=_SC_PARAMS,
        scratch_types=[
            pltpu.VMEM((_CH, _CB), jnp.int32),
            pltpu.VMEM((_CH, _CB), jnp.int32),
            pltpu.VMEM((_CB, 16), jnp.float32),
            pltpu.VMEM((_PPW,), jnp.int32),
            pltpu.VMEM((_PPW, _D), jnp.float32),
            pltpu.VMEM_SHARED((_N, 16), jnp.float32),
            pltpu.VMEM_SHARED((_N, 16), jnp.float32),
            pltpu.SemaphoreType.DMA,
        ],
    )


def _spmm_body(xlo_hbm, xhi_hbm, src_hbm, dst_hbm, zeros_hbm,
               outlo_hbm, outhi_hbm,
               sidx, didx, rows, acc,
               g0, g1, g2, g3, s0, s1, s2, s3):
    cid = lax.axis_index("c")
    sid = lax.axis_index("s")
    pltpu.sync_copy(src_hbm.at[cid, sid], sidx)
    pltpu.sync_copy(dst_hbm.at[sid], didx)
    gsems = (g0, g1, g2, g3)
    ssems = (s0, s1, s2, s3)
    for x_hbm, out_hbm in ((xlo_hbm, outlo_hbm), (xhi_hbm, outhi_hbm)):
        pltpu.sync_copy(zeros_hbm, acc.at[pl.ds(sid * _RPS, _RPS)])
        plsc.subcore_barrier()
        gh = [None] * _NB
        sh = [None] * _NB
        for c in range(_CH2 + 2):
            if c < _CH2:
                i = c % _NB
                if sh[i] is not None:
                    sh[i].wait()
                gh[i] = pltpu.async_copy(x_hbm.at[sidx.at[c]], rows.at[i],
                                         gsems[i])
            if c >= 2:
                cc = c - 2
                j = cc % _NB
                gh[j].wait()
                sh[j] = pltpu.async_copy(rows.at[j], acc.at[didx.at[cc]],
                                         ssems[j], add=True)
        sh[(_CH2 - 2) % _NB].wait()
        sh[(_CH2 - 1) % _NB].wait()
        plsc.subcore_barrier()
        pltpu.sync_copy(acc.at[pl.ds(sid * _RPS, _RPS)],
                        out_hbm.at[cid, pl.ds(sid * _RPS, _RPS)])


@functools.cache
def _spmm():
    return pl.kernel(
        _spmm_body,
        out_type=(
            jax.ShapeDtypeStruct((_NC, _N, _HD), jnp.float32),
            jax.ShapeDtypeStruct((_NC, _N, _HD), jnp.float32),
        ),
        mesh=plsc.VectorSubcoreMesh(core_axis_name="c", subcore_axis_name="s"),
        compiler_params=_SC_PARAMS,
        scratch_types=[
            pltpu.VMEM((_CH2, _CB), jnp.int32),
            pltpu.VMEM((_CH2, _CB), jnp.int32),
            pltpu.VMEM((_NB, _CB, _HD), jnp.float32),
            pltpu.VMEM_SHARED((_N, _HD), jnp.float32),
            pltpu.SemaphoreType.DMA,
            pltpu.SemaphoreType.DMA,
            pltpu.SemaphoreType.DMA,
            pltpu.SemaphoreType.DMA,
            pltpu.SemaphoreType.DMA,
            pltpu.SemaphoreType.DMA,
            pltpu.SemaphoreType.DMA,
            pltpu.SemaphoreType.DMA,
        ],
    )


def _norm(p_ref):
    cnt = p_ref[0, :, 0:1] + p_ref[1, :, 0:1]
    return lax.rsqrt(jnp.maximum(cnt, 1.0))


def _prep_body(feat_ref, fperm_ref, dop_ref, xlo_ref, xhi_ref):
    no = _norm(dop_ref)
    x1 = feat_ref[...] * no
    x2 = fperm_ref[...] * no
    xlo_ref[0, :, :] = x1[:, :_HD]
    xlo_ref[1, :, :] = x2[:, :_HD]
    xhi_ref[0, :, :] = x1[:, _HD:]
    xhi_ref[1, :, :] = x2[:, _HD:]


_prep = pl.pallas_call(
    _prep_body,
    grid=(_GRID,),
    in_specs=[
        pl.BlockSpec((_BLK, _D), lambda i: (i, 0)),
        pl.BlockSpec((_BLK, _D), lambda i: (i, 0)),
        pl.BlockSpec((_NC, _BLK, 16), lambda i: (0, i, 0)),
    ],
    out_specs=[pl.BlockSpec((2, _BLK, _HD), lambda i: (0, i, 0))] * 2,
    out_shape=[jax.ShapeDtypeStruct((2, _N, _HD), jnp.float32)] * 2,
)


def _layer_body(alo_ref, ahi_ref, dop_ref, dip_ref, w_ref, b_ref,
                ylo_ref, yhi_ref):
    ni = _norm(dip_ref)
    no = _norm(dop_ref)
    for e in range(2):
        agg = jnp.concatenate([alo_ref[e, :, :], ahi_ref[e, :, :]], axis=1)
        z = jnp.dot(agg, w_ref[...], preferred_element_type=jnp.float32)
        h = jnp.maximum(z * ni + b_ref[...], 0.0)
        y = h * no
        ylo_ref[e, :, :] = y[:, :_HD]
        yhi_ref[e, :, :] = y[:, _HD:]


_layer = pl.pallas_call(
    _layer_body,
    grid=(_GRID,),
    in_specs=[
        pl.BlockSpec((2, _BLK, _HD), lambda i: (0, i, 0)),
        pl.BlockSpec((2, _BLK, _HD), lambda i: (0, i, 0)),
        pl.BlockSpec((_NC, _BLK, 16), lambda i: (0, i, 0)),
        pl.BlockSpec((_NC, _BLK, 16), lambda i: (0, i, 0)),
        pl.BlockSpec((_D, _D), lambda i: (0, 0)),
        pl.BlockSpec((1, _D), lambda i: (0, 0)),
    ],
    out_specs=[pl.BlockSpec((2, _BLK, _HD), lambda i: (0, i, 0))] * 2,
    out_shape=[jax.ShapeDtypeStruct((2, _N, _HD), jnp.float32)] * 2,
)


def _final_body(alo_ref, ahi_ref, dip_ref, w2_ref, b2_ref, wm_ref, bm_ref,
                lab_ref, out_ref):
    i = pl.program_id(0)

    @pl.when(i == 0)
    def _():
        out_ref[0, 0] = 0.0

    ni = _norm(dip_ref)
    wmv = jnp.sum(wm_ref[...], axis=1, keepdims=True)
    bms = jnp.sum(bm_ref[...])
    total = 0.0
    for e in range(2):
        agg = jnp.concatenate([alo_ref[e, :, :], ahi_ref[e, :, :]], axis=1)
        z = jnp.dot(agg, w2_ref[...], preferred_element_type=jnp.float32)
        h = jnp.maximum(z * ni + b2_ref[...], 0.0)
        s = jnp.dot(h, wmv, preferred_element_type=jnp.float32) + bms
        l = lab_ref[e, :, :]
        bce = jnp.maximum(s, 0.0) - s * l + jnp.log(1.0 + jnp.exp(-jnp.abs(s)))
        total = total + jnp.sum(bce)
    out_ref[0, 0] = out_ref[0, 0] + total / (2.0 * _N)


_final = pl.pallas_call(
    _final_body,
    grid=(_GRID,),
    in_specs=[
        pl.BlockSpec((2, _BLK, _HD), lambda i: (0, i, 0)),
        pl.BlockSpec((2, _BLK, _HD), lambda i: (0, i, 0)),
        pl.BlockSpec((_NC, _BLK, 16), lambda i: (0, i, 0)),
        pl.BlockSpec((_D, _D), lambda i: (0, 0)),
        pl.BlockSpec((1, _D), lambda i: (0, 0)),
        pl.BlockSpec((_D, _D), lambda i: (0, 0)),
        pl.BlockSpec((1, _D), lambda i: (0, 0)),
        pl.BlockSpec((2, _BLK, 1), lambda i: (0, i, 0)),
    ],
    out_specs=pl.BlockSpec(memory_space=pltpu.SMEM),
    out_shape=jax.ShapeDtypeStruct((1, 1), jnp.float32),
)


def kernel(features, edge_index, perm, labels, W1, b1, W2, b2, Wm, bm):
    src0 = edge_index[0].reshape(_NS, _CH2, _CB)
    dst2 = edge_index[1].reshape(_NS, _CH2, _CB)
    srcb = jnp.stack([src0, src0 + _N])
    src3 = edge_index[0].reshape(_NW, _CH, _CB)
    dst3 = edge_index[1].reshape(_NW, _CH, _CB)
    perm2 = jnp.concatenate(
        [perm, jnp.zeros((_PN - _N,), jnp.int32)]).reshape(_NW, _PPW)
    zeros64 = jnp.zeros((_RPS, _HD), jnp.float32)
    zeros16 = jnp.zeros((_RPS, 16), jnp.float32)
    ones16 = jnp.ones((_CB, 16), jnp.float32)
    b1r = b1.reshape(1, _D)
    b2r = b2.reshape(1, _D)
    bmr = bm.reshape(1, _D)
    lab2 = labels.reshape(2, _N, 1)

    deg_k = _deg()
    spmm_k = _spmm()
    dout_p, din_p, fperm = deg_k(src3, dst3, perm2, features, ones16, zeros16)
    xlo, xhi = _prep(features, fperm, dout_p)
    alo, ahi = spmm_k(xlo.reshape(2 * _N, _HD), xhi.reshape(2 * _N, _HD),
                      srcb, dst2, zeros64)
    ylo, yhi = _layer(alo, ahi, dout_p, din_p, W1, b1r)
    glo, ghi = spmm_k(ylo.reshape(2 * _N, _HD), yhi.reshape(2 * _N, _HD),
                      srcb, dst2, zeros64)
    out = _final(glo, ghi, din_p, W2, b2r, Wm, bmr, lab2)
    return out[0, 0]
